# baseline (speedup 1.0000x reference)
import jax
import jax.numpy as jnp
import numpy as np
from jax.experimental import pallas as pl
from jax.experimental.pallas import tpu as pltpu

ENC_NH = 256
DEC_NH = 256
Z_NH = 32

bf16 = jnp.bfloat16
f32 = jnp.float32


def _dot(a, b):
    return jax.lax.dot_general(a, b, (((1,), (0,)), ((), ())),
                               preferred_element_type=f32)


def _gru_gates(gi, gh, h, nh):
    r = jax.nn.sigmoid(gi[:, :nh] + gh[:, :nh])
    u = jax.nn.sigmoid(gi[:, nh:2 * nh] + gh[:, nh:2 * nh])
    n = jnp.tanh(gi[:, 2 * nh:] + r * gh[:, 2 * nh:])
    return (1.0 - u) * n + u * h


def _onehot16(x_col, vocab):
    iot = jax.lax.broadcasted_iota(jnp.int32, (x_col.shape[0], vocab), 1)
    return (x_col == iot).astype(bf16)


def _embed16(x_col, emb_ref):
    oh = _onehot16(x_col, emb_ref.shape[0])
    return _dot(oh, emb_ref[...]).astype(bf16)


def _enc_kernel(T, V, B):
    def body(xf_ref, xb_ref, emb_ref, WihfT_ref, WihbT_ref, bihf_ref,
             bihb_ref, WhhfT_ref, WhhbT_ref, bhhf_ref, bhhb_ref,
             hf_out, hb_out, hf_s, hb_s):
        t = pl.program_id(0)

        @pl.when(t == 0)
        def _init():
            hf_s[...] = jnp.zeros_like(hf_s)
            hb_s[...] = jnp.zeros_like(hb_s)

        def step(x_ref, WihT_ref, bih_ref, WhhT_ref, bhh_ref, h_s):
            e16 = _embed16(x_ref[0], emb_ref)
            gi = _dot(e16, WihT_ref[...]) + bih_ref[...]
            gh = _dot(h_s[...].astype(bf16), WhhT_ref[...]) + bhh_ref[...]
            h_s[...] = _gru_gates(gi, gh, h_s[...], ENC_NH)

        step(xf_ref, WihfT_ref, bihf_ref, WhhfT_ref, bhhf_ref, hf_s)
        step(xb_ref, WihbT_ref, bihb_ref, WhhbT_ref, bhhb_ref, hb_s)

        @pl.when(t == T - 1)
        def _fin():
            hf_out[...] = hf_s[...]
            hb_out[...] = hb_s[...]

    return body


def _vq_kernel(K, B):
    def body(hf_ref, hb_ref, dWT_ref, db_ref, cb_ref, cbT_ref,
             upWT_ref, upb_ref,
             closest_out, q_out, hidden_out, zq_out):
        cat16 = jnp.concatenate([hf_ref[...], hb_ref[...]], axis=1).astype(bf16)
        z = _dot(cat16, dWT_ref[...]) + db_ref[...]
        zsq = jnp.sum(z * z, axis=1, keepdims=True)
        cbT = cbT_ref[...]
        csq = jnp.sum(cbT * cbT, axis=0, keepdims=True)
        mm = _dot(z.astype(bf16), cbT.astype(bf16))
        score = (zsq + csq) - 2.0 * mm
        m = jnp.min(score, axis=1, keepdims=True)
        iot = jax.lax.broadcasted_iota(jnp.int32, (B, K), 1)
        idx = jnp.min(jnp.where(score == m, iot, K), axis=1, keepdims=True)
        closest_out[...] = idx
        ohK = (iot == idx).astype(bf16)
        quant = _dot(ohK, cb_ref[...].astype(bf16))
        diff = quant - z
        qrow = jnp.mean(diff * diff, axis=1)
        qv = jnp.sum(qrow * 0.1 + qrow * 0.1)
        q_out[...] = jnp.reshape(qv, (1, 1))
        hidden_out[...] = _dot(quant.astype(bf16), upWT_ref[...]) + upb_ref[...]
        zq_out[...] = quant

    return body


def _dec_kernel(Tm1, V, B):
    def body(src_ref, tgt_ref, demb_ref, WihT_ref, bih_ref, zq_ref,
             WhhT_ref, bhh_ref, outWT_ref, outb_ref, h0_ref,
             pred_out, xl_out, np_out, h_s, xl_s, np_s):
        t = pl.program_id(0)

        @pl.when(t == 0)
        def _init():
            h_s[...] = h0_ref[...]
            xl_s[0, 0] = 0.0
            np_s[0, 0] = 0

        e16 = _embed16(src_ref[0], demb_ref)
        cat16 = jnp.concatenate([zq_ref[...].astype(bf16), e16], axis=1)
        gi = _dot(cat16, WihT_ref[...]) + bih_ref[...]
        gh = _dot(h_s[...].astype(bf16), WhhT_ref[...]) + bhh_ref[...]
        h_new = _gru_gates(gi, gh, h_s[...], DEC_NH)
        h_s[...] = h_new

        logits = _dot(h_new.astype(bf16), outWT_ref[...]) + outb_ref[...]
        mx = jnp.max(logits, axis=1, keepdims=True)
        shifted = logits - mx
        lsum = jnp.log(jnp.sum(jnp.exp(shifted), axis=1, keepdims=True))
        tgt = tgt_ref[0]
        iot = jax.lax.broadcasted_iota(jnp.int32, (B, V), 1)
        sh_tgt = jnp.sum(jnp.where(tgt == iot, shifted, 0.0),
                         axis=1, keepdims=True)
        nll = lsum - sh_tgt
        mask = (tgt != 0)
        xl_s[0, 0] += jnp.sum(jnp.where(mask, nll, 0.0))
        np_s[0, 0] += jnp.sum(mask.astype(jnp.int32))

        pidx = jnp.min(jnp.where(logits == mx, iot, V), axis=1, keepdims=True)
        pred_out[0] = pidx

        @pl.when(t == Tm1 - 1)
        def _fin():
            xl_out[...] = jnp.reshape(xl_s[0, 0], (1, 1))
            np_out[...] = jnp.reshape(np_s[0, 0], (1, 1))

    return body


def _forward(x, enc_emb, enc_Wih_f, enc_Whh_f, enc_bih_f, enc_bhh_f,
             enc_Wih_b, enc_Whh_b, enc_bih_b, enc_bhh_b, down_W, down_b,
             codebook, up_W, up_b, dec_emb, dec_Wih, dec_Whh, dec_bih,
             dec_bhh, out_W, out_b):
    B, T = x.shape
    V = enc_emb.shape[0]
    E = enc_emb.shape[1]
    K = codebook.shape[0]
    Tm1 = T - 1

    xT = x.T.reshape(T, B, 1)
    arb = pltpu.CompilerParams(dimension_semantics=("arbitrary",))
    row = lambda v: v.reshape(1, -1)

    hf, hb = pl.pallas_call(
        _enc_kernel(T, V, B),
        grid=(T,),
        in_specs=[
            pl.BlockSpec((1, B, 1), lambda t: (t, 0, 0)),
            pl.BlockSpec((1, B, 1), lambda t: (T - 1 - t, 0, 0)),
            pl.BlockSpec((V, E), lambda t: (0, 0)),
            pl.BlockSpec((E, 3 * ENC_NH), lambda t: (0, 0)),
            pl.BlockSpec((E, 3 * ENC_NH), lambda t: (0, 0)),
            pl.BlockSpec((1, 3 * ENC_NH), lambda t: (0, 0)),
            pl.BlockSpec((1, 3 * ENC_NH), lambda t: (0, 0)),
            pl.BlockSpec((ENC_NH, 3 * ENC_NH), lambda t: (0, 0)),
            pl.BlockSpec((ENC_NH, 3 * ENC_NH), lambda t: (0, 0)),
            pl.BlockSpec((1, 3 * ENC_NH), lambda t: (0, 0)),
            pl.BlockSpec((1, 3 * ENC_NH), lambda t: (0, 0)),
        ],
        out_specs=[
            pl.BlockSpec((B, ENC_NH), lambda t: (0, 0)),
            pl.BlockSpec((B, ENC_NH), lambda t: (0, 0)),
        ],
        out_shape=[
            jax.ShapeDtypeStruct((B, ENC_NH), f32),
            jax.ShapeDtypeStruct((B, ENC_NH), f32),
        ],
        scratch_shapes=[
            pltpu.VMEM((B, ENC_NH), f32),
            pltpu.VMEM((B, ENC_NH), f32),
        ],
        compiler_params=arb,
    )(
        xT, xT, enc_emb.astype(bf16),
        enc_Wih_f.T.astype(bf16), enc_Wih_b.T.astype(bf16),
        row(enc_bih_f), row(enc_bih_b),
        enc_Whh_f.T.astype(bf16), enc_Whh_b.T.astype(bf16),
        row(enc_bhh_f), row(enc_bhh_b),
    )

    closest, q, hidden, zq = pl.pallas_call(
        _vq_kernel(K, B),
        out_shape=[
            jax.ShapeDtypeStruct((B, 1), jnp.int32),
            jax.ShapeDtypeStruct((1, 1), f32),
            jax.ShapeDtypeStruct((B, DEC_NH), f32),
            jax.ShapeDtypeStruct((B, Z_NH), f32),
        ],
    )(
        hf, hb,
        down_W.T.astype(bf16), row(down_b),
        codebook, codebook.T,
        up_W.T.astype(bf16), row(up_b),
    )

    predT, xl, npad = pl.pallas_call(
        _dec_kernel(Tm1, V, B),
        grid=(Tm1,),
        in_specs=[
            pl.BlockSpec((1, B, 1), lambda t: (t, 0, 0)),
            pl.BlockSpec((1, B, 1), lambda t: (t, 0, 0)),
            pl.BlockSpec((V, E), lambda t: (0, 0)),
            pl.BlockSpec((Z_NH + E, 3 * DEC_NH), lambda t: (0, 0)),
            pl.BlockSpec((1, 3 * DEC_NH), lambda t: (0, 0)),
            pl.BlockSpec((B, Z_NH), lambda t: (0, 0)),
            pl.BlockSpec((DEC_NH, 3 * DEC_NH), lambda t: (0, 0)),
            pl.BlockSpec((1, 3 * DEC_NH), lambda t: (0, 0)),
            pl.BlockSpec((DEC_NH, V), lambda t: (0, 0)),
            pl.BlockSpec((1, V), lambda t: (0, 0)),
            pl.BlockSpec((B, DEC_NH), lambda t: (0, 0)),
        ],
        out_specs=[
            pl.BlockSpec((1, B, 1), lambda t: (t, 0, 0)),
            pl.BlockSpec((1, 1), lambda t: (0, 0)),
            pl.BlockSpec((1, 1), lambda t: (0, 0)),
        ],
        out_shape=[
            jax.ShapeDtypeStruct((Tm1, B, 1), jnp.int32),
            jax.ShapeDtypeStruct((1, 1), f32),
            jax.ShapeDtypeStruct((1, 1), jnp.int32),
        ],
        scratch_shapes=[
            pltpu.VMEM((B, DEC_NH), f32),
            pltpu.SMEM((1, 1), f32),
            pltpu.SMEM((1, 1), jnp.int32),
        ],
        compiler_params=arb,
    )(
        xT[:Tm1], xT[1:], dec_emb.astype(bf16),
        dec_Wih.T.astype(bf16), row(dec_bih), zq,
        dec_Whh.T.astype(bf16), row(dec_bhh),
        out_W.T.astype(bf16), row(out_b),
        hidden,
    )

    return xl, npad, predT, q, closest


def kernel(x, enc_emb, enc_Wih_f, enc_Whh_f, enc_bih_f, enc_bhh_f,
           enc_Wih_b, enc_Whh_b, enc_bih_b, enc_bhh_b, down_W, down_b,
           codebook, up_W, up_b, dec_emb, dec_Wih, dec_Whh, dec_bih,
           dec_bhh, out_W, out_b):
    args = (x, enc_emb, enc_Wih_f, enc_Whh_f, enc_bih_f, enc_bhh_f,
            enc_Wih_b, enc_Whh_b, enc_bih_b, enc_bhh_b, down_W, down_b,
            codebook, up_W, up_b, dec_emb, dec_Wih, dec_Whh, dec_bih,
            dec_bhh, out_W, out_b)
    devs = jax.devices()
    B = x.shape[0]
    if len(devs) >= 2 and B % 2 == 0:
        P = jax.sharding.PartitionSpec
        mesh = jax.sharding.Mesh(np.asarray(devs[:2]), ("b",))

        xl, npad, predT, q, closest = jax.shard_map(
            _forward, mesh=mesh, check_vma=False,
            in_specs=(P("b"),) + (P(),) * 21,
            out_specs=(P("b"), P("b"), P(None, "b", None), P("b"),
                       P("b", None)),
        )(*args)
        xl = jnp.sum(xl, axis=0, keepdims=True)
        npad = jnp.sum(npad, axis=0, keepdims=True)
        q = jnp.sum(q, axis=0, keepdims=True)
    else:
        xl, npad, predT, q, closest = _forward(*args)
    xloss = xl[0, 0]
    nonpadded = npad[0, 0]
    pred = predT[:, :, 0].T
    Q = q[0, 0]
    return xloss, nonpadded, pred, Q, closest


# speedup vs baseline: 1.9301x; 1.9301x over previous
import jax
import jax.numpy as jnp
import numpy as np
from jax.experimental import pallas as pl
from jax.experimental.pallas import tpu as pltpu

ENC_NH = 256
DEC_NH = 256
Z_NH = 32

bf16 = jnp.bfloat16
f32 = jnp.float32


def _dot(a, b):
    return jax.lax.dot_general(a, b, (((1,), (0,)), ((), ())),
                               preferred_element_type=f32)


def _sigmoid(x):
    return 0.5 + 0.5 * jnp.tanh(0.5 * x)


def _gru_gates(gi, gh, h, nh):
    r = _sigmoid(gi[:, :nh] + gh[:, :nh])
    u = _sigmoid(gi[:, nh:2 * nh] + gh[:, nh:2 * nh])
    n = jnp.tanh(gi[:, 2 * nh:] + r * gh[:, 2 * nh:])
    return (1.0 - u) * n + u * h


def _onehot16(x_col, vocab):
    iot = jax.lax.broadcasted_iota(jnp.int32, (x_col.shape[0], vocab), 1)
    return (x_col == iot).astype(bf16)


def _embed16(x_col, emb_ref):
    oh = _onehot16(x_col, emb_ref.shape[0])
    return _dot(oh, emb_ref[...]).astype(bf16)


def _enc_kernel(T, V, B):
    def body(xf_ref, xb_ref, emb_ref, WihfT_ref, WihbT_ref, bihf_ref,
             bihb_ref, WhhfT_ref, WhhbT_ref, bhhf_ref, bhhb_ref,
             hf_out, hb_out, hf_s, hb_s):
        t = pl.program_id(0)

        @pl.when(t == 0)
        def _init():
            hf_s[...] = jnp.zeros_like(hf_s)
            hb_s[...] = jnp.zeros_like(hb_s)

        def step(x_ref, WihT_ref, bih_ref, WhhT_ref, bhh_ref, h_s):
            e16 = _embed16(x_ref[0], emb_ref)
            gi = _dot(e16, WihT_ref[...]) + bih_ref[...]
            gh = _dot(h_s[...].astype(bf16), WhhT_ref[...]) + bhh_ref[...]
            h_s[...] = _gru_gates(gi, gh, h_s[...], ENC_NH)

        step(xf_ref, WihfT_ref, bihf_ref, WhhfT_ref, bhhf_ref, hf_s)
        step(xb_ref, WihbT_ref, bihb_ref, WhhbT_ref, bhhb_ref, hb_s)

        @pl.when(t == T - 1)
        def _fin():
            hf_out[...] = hf_s[...]
            hb_out[...] = hb_s[...]

    return body


def _vq_kernel(K, B):
    def body(hf_ref, hb_ref, dWT_ref, db_ref, cb_ref, cbT_ref,
             upWT_ref, upb_ref,
             closest_out, q_out, hidden_out, zq_out):
        cat16 = jnp.concatenate([hf_ref[...], hb_ref[...]], axis=1).astype(bf16)
        z = _dot(cat16, dWT_ref[...]) + db_ref[...]
        zsq = jnp.sum(z * z, axis=1, keepdims=True)
        cbT = cbT_ref[...]
        csq = jnp.sum(cbT * cbT, axis=0, keepdims=True)
        mm = _dot(z.astype(bf16), cbT.astype(bf16))
        score = (zsq + csq) - 2.0 * mm
        m = jnp.min(score, axis=1, keepdims=True)
        iot = jax.lax.broadcasted_iota(jnp.int32, (B, K), 1)
        idx = jnp.min(jnp.where(score == m, iot, K), axis=1, keepdims=True)
        closest_out[...] = idx
        ohK = (iot == idx).astype(bf16)
        quant = _dot(ohK, cb_ref[...].astype(bf16))
        diff = quant - z
        qrow = jnp.mean(diff * diff, axis=1)
        qv = jnp.sum(qrow * 0.1 + qrow * 0.1)
        q_out[...] = jnp.reshape(qv, (1, 1))
        hidden_out[...] = _dot(quant.astype(bf16), upWT_ref[...]) + upb_ref[...]
        zq_out[...] = quant

    return body


def _dec_kernel(Tm1, V, B):
    def body(src_ref, tgt_ref, demb_ref, WihT_ref, bih_ref, zq_ref,
             WhhT_ref, bhh_ref, outWT_ref, outb_ref, h0_ref,
             pred_out, xl_out, np_out, h_s, xl_s, np_s):
        t = pl.program_id(0)

        @pl.when(t == 0)
        def _init():
            h_s[...] = h0_ref[...]
            xl_s[0, 0] = 0.0
            np_s[0, 0] = 0

        e16 = _embed16(src_ref[0], demb_ref)
        cat16 = jnp.concatenate([zq_ref[...].astype(bf16), e16], axis=1)
        gi = _dot(cat16, WihT_ref[...]) + bih_ref[...]
        gh = _dot(h_s[...].astype(bf16), WhhT_ref[...]) + bhh_ref[...]
        h_new = _gru_gates(gi, gh, h_s[...], DEC_NH)
        h_s[...] = h_new

        logits = _dot(h_new.astype(bf16), outWT_ref[...]) + outb_ref[...]
        mx = jnp.max(logits, axis=1, keepdims=True)
        shifted = logits - mx
        lsum = jnp.log(jnp.sum(jnp.exp(shifted), axis=1, keepdims=True))
        tgt = tgt_ref[0]
        iot = jax.lax.broadcasted_iota(jnp.int32, (B, V), 1)
        sh_tgt = jnp.sum(jnp.where(tgt == iot, shifted, 0.0),
                         axis=1, keepdims=True)
        nll = lsum - sh_tgt
        mask = (tgt != 0)
        xl_s[0, 0] += jnp.sum(jnp.where(mask, nll, 0.0))
        np_s[0, 0] += jnp.sum(mask.astype(jnp.int32))

        pidx = jnp.min(jnp.where(logits == mx, iot, V), axis=1, keepdims=True)
        pred_out[0] = pidx

        @pl.when(t == Tm1 - 1)
        def _fin():
            xl_out[...] = jnp.reshape(xl_s[0, 0], (1, 1))
            np_out[...] = jnp.reshape(np_s[0, 0], (1, 1))

    return body


def _forward(x, enc_emb, enc_Wih_f, enc_Whh_f, enc_bih_f, enc_bhh_f,
             enc_Wih_b, enc_Whh_b, enc_bih_b, enc_bhh_b, down_W, down_b,
             codebook, up_W, up_b, dec_emb, dec_Wih, dec_Whh, dec_bih,
             dec_bhh, out_W, out_b):
    B, T = x.shape
    V = enc_emb.shape[0]
    E = enc_emb.shape[1]
    K = codebook.shape[0]
    Tm1 = T - 1

    xT = x.T.reshape(T, B, 1)
    arb = pltpu.CompilerParams(dimension_semantics=("arbitrary",))
    row = lambda v: v.reshape(1, -1)

    hf, hb = pl.pallas_call(
        _enc_kernel(T, V, B),
        grid=(T,),
        in_specs=[
            pl.BlockSpec((1, B, 1), lambda t: (t, 0, 0)),
            pl.BlockSpec((1, B, 1), lambda t: (T - 1 - t, 0, 0)),
            pl.BlockSpec((V, E), lambda t: (0, 0)),
            pl.BlockSpec((E, 3 * ENC_NH), lambda t: (0, 0)),
            pl.BlockSpec((E, 3 * ENC_NH), lambda t: (0, 0)),
            pl.BlockSpec((1, 3 * ENC_NH), lambda t: (0, 0)),
            pl.BlockSpec((1, 3 * ENC_NH), lambda t: (0, 0)),
            pl.BlockSpec((ENC_NH, 3 * ENC_NH), lambda t: (0, 0)),
            pl.BlockSpec((ENC_NH, 3 * ENC_NH), lambda t: (0, 0)),
            pl.BlockSpec((1, 3 * ENC_NH), lambda t: (0, 0)),
            pl.BlockSpec((1, 3 * ENC_NH), lambda t: (0, 0)),
        ],
        out_specs=[
            pl.BlockSpec((B, ENC_NH), lambda t: (0, 0)),
            pl.BlockSpec((B, ENC_NH), lambda t: (0, 0)),
        ],
        out_shape=[
            jax.ShapeDtypeStruct((B, ENC_NH), f32),
            jax.ShapeDtypeStruct((B, ENC_NH), f32),
        ],
        scratch_shapes=[
            pltpu.VMEM((B, ENC_NH), f32),
            pltpu.VMEM((B, ENC_NH), f32),
        ],
        compiler_params=arb,
    )(
        xT, xT, enc_emb.astype(bf16),
        enc_Wih_f.T.astype(bf16), enc_Wih_b.T.astype(bf16),
        row(enc_bih_f), row(enc_bih_b),
        enc_Whh_f.T.astype(bf16), enc_Whh_b.T.astype(bf16),
        row(enc_bhh_f), row(enc_bhh_b),
    )

    closest, q, hidden, zq = pl.pallas_call(
        _vq_kernel(K, B),
        out_shape=[
            jax.ShapeDtypeStruct((B, 1), jnp.int32),
            jax.ShapeDtypeStruct((1, 1), f32),
            jax.ShapeDtypeStruct((B, DEC_NH), f32),
            jax.ShapeDtypeStruct((B, Z_NH), f32),
        ],
    )(
        hf, hb,
        down_W.T.astype(bf16), row(down_b),
        codebook, codebook.T,
        up_W.T.astype(bf16), row(up_b),
    )

    predT, xl, npad = pl.pallas_call(
        _dec_kernel(Tm1, V, B),
        grid=(Tm1,),
        in_specs=[
            pl.BlockSpec((1, B, 1), lambda t: (t, 0, 0)),
            pl.BlockSpec((1, B, 1), lambda t: (t, 0, 0)),
            pl.BlockSpec((V, E), lambda t: (0, 0)),
            pl.BlockSpec((Z_NH + E, 3 * DEC_NH), lambda t: (0, 0)),
            pl.BlockSpec((1, 3 * DEC_NH), lambda t: (0, 0)),
            pl.BlockSpec((B, Z_NH), lambda t: (0, 0)),
            pl.BlockSpec((DEC_NH, 3 * DEC_NH), lambda t: (0, 0)),
            pl.BlockSpec((1, 3 * DEC_NH), lambda t: (0, 0)),
            pl.BlockSpec((DEC_NH, V), lambda t: (0, 0)),
            pl.BlockSpec((1, V), lambda t: (0, 0)),
            pl.BlockSpec((B, DEC_NH), lambda t: (0, 0)),
        ],
        out_specs=[
            pl.BlockSpec((1, B, 1), lambda t: (t, 0, 0)),
            pl.BlockSpec((1, 1), lambda t: (0, 0)),
            pl.BlockSpec((1, 1), lambda t: (0, 0)),
        ],
        out_shape=[
            jax.ShapeDtypeStruct((Tm1, B, 1), jnp.int32),
            jax.ShapeDtypeStruct((1, 1), f32),
            jax.ShapeDtypeStruct((1, 1), jnp.int32),
        ],
        scratch_shapes=[
            pltpu.VMEM((B, DEC_NH), f32),
            pltpu.SMEM((1, 1), f32),
            pltpu.SMEM((1, 1), jnp.int32),
        ],
        compiler_params=arb,
    )(
        xT[:Tm1], xT[1:], dec_emb.astype(bf16),
        dec_Wih.T.astype(bf16), row(dec_bih), zq,
        dec_Whh.T.astype(bf16), row(dec_bhh),
        out_W.T.astype(bf16), row(out_b),
        hidden,
    )

    return xl, npad, predT, q, closest


def kernel(x, enc_emb, enc_Wih_f, enc_Whh_f, enc_bih_f, enc_bhh_f,
           enc_Wih_b, enc_Whh_b, enc_bih_b, enc_bhh_b, down_W, down_b,
           codebook, up_W, up_b, dec_emb, dec_Wih, dec_Whh, dec_bih,
           dec_bhh, out_W, out_b):
    args = (x, enc_emb, enc_Wih_f, enc_Whh_f, enc_bih_f, enc_bhh_f,
            enc_Wih_b, enc_Whh_b, enc_bih_b, enc_bhh_b, down_W, down_b,
            codebook, up_W, up_b, dec_emb, dec_Wih, dec_Whh, dec_bih,
            dec_bhh, out_W, out_b)
    xl, npad, predT, q, closest = _forward(*args)
    xloss = xl[0, 0]
    nonpadded = npad[0, 0]
    pred = predT[:, :, 0].T
    Q = q[0, 0]
    return xloss, nonpadded, pred, Q, closest
